# baseline (speedup 1.0000x reference)
import jax
import jax.numpy as jnp
from jax import lax
from jax.experimental import pallas as pl
from jax.experimental.pallas import tpu as pltpu
from jax.experimental.pallas import tpu_sc as plsc

NUM_ENT = 1000000
EMB_DIM = 64
BATCH = 16384
NC = 2
NS = 16
NW = NC * NS
BPW = BATCH // NW
NIDS = 5
CHUNK = 128
NCHUNK = NIDS * BPW // CHUNK
RPW = BPW // CHUNK

LBLK = 131072
NJT = EMB_DIM // 8


def _tc_body(x_ref, v_ref, acc_s, acc_q):
    j = pl.program_id(1)
    x = x_ref[...]
    s = jnp.sum(x, axis=0)
    q = jnp.sum(x * x, axis=0)

    @pl.when(j == 0)
    def _():
        acc_s[...] = s
        acc_q[...] = q

    @pl.when(j > 0)
    def _():
        acc_s[...] += s
        acc_q[...] += q

    @pl.when(j == NJT - 1)
    def _():
        v_ref[...] = acc_s[...] * lax.rsqrt(acc_q[...])


def _sc_body(ids_t_hbm, ids_f_hbm, v_hbm, out_t, out_hf, out_tf,
             idx_v, s_v, ot_v, ohf_v, otf_v, sem, sem2):
    w = lax.axis_index("s") * NC + lax.axis_index("c")

    staged = []
    for k in range(3):
        staged.append(pltpu.async_copy(ids_t_hbm.at[k, pl.ds(w * RPW, RPW)],
                                       idx_v.at[pl.ds(k * RPW, RPW)], sem2))
    for k in range(2):
        staged.append(pltpu.async_copy(ids_f_hbm.at[k, pl.ds(w * RPW, RPW)],
                                       idx_v.at[pl.ds((3 + k) * RPW, RPW)],
                                       sem2))
    for cp in staged:
        cp.wait()

    gathers = [
        pltpu.async_copy(v_hbm.at[idx_v.at[c]],
                         s_v.at[pl.ds(c * CHUNK, CHUNK)], sem)
        for c in range(NCHUNK)
    ]
    for cp in gathers:
        cp.wait()

    def comb_body(i, carry):
        o = i * 16
        sh = s_v[pl.ds(o, 16)]
        sr = s_v[pl.ds(BPW + o, 16)]
        st = s_v[pl.ds(2 * BPW + o, 16)]
        shf = s_v[pl.ds(3 * BPW + o, 16)]
        stf = s_v[pl.ds(4 * BPW + o, 16)]
        ot_v[pl.ds(o, 16)] = sh + sr - st
        ohf_v[pl.ds(o, 16)] = shf + sr - st
        otf_v[pl.ds(o, 16)] = sh + sr - stf
        return carry

    lax.fori_loop(0, BPW // 16, comb_body, 0)

    base = w * BPW
    pltpu.sync_copy(ot_v, out_t.at[pl.ds(base, BPW)])
    pltpu.sync_copy(ohf_v, out_hf.at[pl.ds(base, BPW)])
    pltpu.sync_copy(otf_v, out_tf.at[pl.ds(base, BPW)])


def kernel(ids_true_batch, ids_false_batch, ent_table):
    tbl_t = ent_table.T

    grid = pl.cdiv(NUM_ENT, LBLK)
    v = pl.pallas_call(
        _tc_body,
        grid=(grid, NJT),
        in_specs=[pl.BlockSpec((8, LBLK), lambda i, j: (j, i))],
        out_specs=pl.BlockSpec((LBLK,), lambda i, j: (i,)),
        out_shape=jax.ShapeDtypeStruct((NUM_ENT,), jnp.float32),
        scratch_shapes=[
            pltpu.VMEM((LBLK,), jnp.float32),
            pltpu.VMEM((LBLK,), jnp.float32),
        ],
    )(tbl_t)

    ids_t = ids_true_batch.astype(jnp.int32).reshape(3, BATCH // CHUNK, CHUNK)
    ids_f = ids_false_batch.astype(jnp.int32).reshape(2, BATCH // CHUNK, CHUNK)

    mesh = plsc.VectorSubcoreMesh(core_axis_name="c", subcore_axis_name="s")
    fn = pl.kernel(
        _sc_body,
        mesh=mesh,
        compiler_params=pltpu.CompilerParams(
            needs_layout_passes=False, use_tc_tiling_on_sc=False
        ),
        out_type=[jax.ShapeDtypeStruct((BATCH,), jnp.float32)] * 3,
        scratch_types=[
            pltpu.VMEM((NCHUNK, CHUNK), jnp.int32),
            pltpu.VMEM((NIDS * BPW,), jnp.float32),
            pltpu.VMEM((BPW,), jnp.float32),
            pltpu.VMEM((BPW,), jnp.float32),
            pltpu.VMEM((BPW,), jnp.float32),
            pltpu.SemaphoreType.DMA,
            pltpu.SemaphoreType.DMA,
        ],
    )
    t, hf, tf = fn(ids_t, ids_f, v)
    return (t, hf, tf)


# speedup vs baseline: 1.7502x; 1.7502x over previous
import jax
import jax.numpy as jnp
from jax import lax
from jax.experimental import pallas as pl
from jax.experimental.pallas import tpu as pltpu
from jax.experimental.pallas import tpu_sc as plsc

NUM_ENT = 1000000
EMB_DIM = 64
BATCH = 16384
NC = 2
NS = 16
NW = NC * NS
BPW = BATCH // NW
NIDS = 5
CHUNK = 128
NCHUNK = NIDS * BPW // CHUNK
RPW = BPW // CHUNK

LBLK = 49152


def _tc_body(x_ref, v_ref):
    x = x_ref[...]
    s = jnp.sum(x, axis=0)
    q = jnp.sum(x * x, axis=0)
    v_ref[...] = s * lax.rsqrt(q)


def _sc_body(ids_t_hbm, ids_f_hbm, v_hbm, out_t, out_hf, out_tf,
             idx_v, s_v, ot_v, ohf_v, otf_v, sem, sem2):
    w = lax.axis_index("s") * NC + lax.axis_index("c")

    staged = []
    for k in range(3):
        staged.append(pltpu.async_copy(ids_t_hbm.at[k, pl.ds(w * RPW, RPW)],
                                       idx_v.at[pl.ds(k * RPW, RPW)], sem2))
    for k in range(2):
        staged.append(pltpu.async_copy(ids_f_hbm.at[k, pl.ds(w * RPW, RPW)],
                                       idx_v.at[pl.ds((3 + k) * RPW, RPW)],
                                       sem2))
    for cp in staged:
        cp.wait()

    gathers = [
        pltpu.async_copy(v_hbm.at[idx_v.at[c]],
                         s_v.at[pl.ds(c * CHUNK, CHUNK)], sem)
        for c in range(NCHUNK)
    ]
    for cp in gathers:
        cp.wait()

    def comb_body(i, carry):
        o = i * 16
        sh = s_v[pl.ds(o, 16)]
        sr = s_v[pl.ds(BPW + o, 16)]
        st = s_v[pl.ds(2 * BPW + o, 16)]
        shf = s_v[pl.ds(3 * BPW + o, 16)]
        stf = s_v[pl.ds(4 * BPW + o, 16)]
        ot_v[pl.ds(o, 16)] = sh + sr - st
        ohf_v[pl.ds(o, 16)] = shf + sr - st
        otf_v[pl.ds(o, 16)] = sh + sr - stf
        return carry

    lax.fori_loop(0, BPW // 16, comb_body, 0)

    base = w * BPW
    pltpu.sync_copy(ot_v, out_t.at[pl.ds(base, BPW)])
    pltpu.sync_copy(ohf_v, out_hf.at[pl.ds(base, BPW)])
    pltpu.sync_copy(otf_v, out_tf.at[pl.ds(base, BPW)])


def kernel(ids_true_batch, ids_false_batch, ent_table):
    tbl_t = ent_table.T

    grid = pl.cdiv(NUM_ENT, LBLK)
    v = pl.pallas_call(
        _tc_body,
        grid=(grid,),
        in_specs=[pl.BlockSpec((EMB_DIM, LBLK), lambda i: (0, i))],
        out_specs=pl.BlockSpec((LBLK,), lambda i: (i,)),
        out_shape=jax.ShapeDtypeStruct((NUM_ENT,), jnp.float32),
    )(tbl_t)

    ids_t = ids_true_batch.astype(jnp.int32).reshape(3, BATCH // CHUNK, CHUNK)
    ids_f = ids_false_batch.astype(jnp.int32).reshape(2, BATCH // CHUNK, CHUNK)

    mesh = plsc.VectorSubcoreMesh(core_axis_name="c", subcore_axis_name="s")
    fn = pl.kernel(
        _sc_body,
        mesh=mesh,
        compiler_params=pltpu.CompilerParams(
            needs_layout_passes=False, use_tc_tiling_on_sc=False
        ),
        out_type=[jax.ShapeDtypeStruct((BATCH,), jnp.float32)] * 3,
        scratch_types=[
            pltpu.VMEM((NCHUNK, CHUNK), jnp.int32),
            pltpu.VMEM((NIDS * BPW,), jnp.float32),
            pltpu.VMEM((BPW,), jnp.float32),
            pltpu.VMEM((BPW,), jnp.float32),
            pltpu.VMEM((BPW,), jnp.float32),
            pltpu.SemaphoreType.DMA,
            pltpu.SemaphoreType.DMA,
        ],
    )
    t, hf, tf = fn(ids_t, ids_f, v)
    return (t, hf, tf)


# speedup vs baseline: 1.7576x; 1.0042x over previous
import jax
import jax.numpy as jnp
from jax import lax
from jax.experimental import pallas as pl
from jax.experimental.pallas import tpu as pltpu
from jax.experimental.pallas import tpu_sc as plsc

NUM_ENT = 1000000
EMB_DIM = 64
BATCH = 16384
NC = 2
NS = 16
NW = NC * NS
BPW = BATCH // NW
NIDS = 5
CHUNK = 128
NCHUNK = NIDS * BPW // CHUNK
RPW = BPW // CHUNK

LBLK = 40960


def _tc_body(x_ref, v_ref):
    x = x_ref[...]
    s = jnp.sum(x, axis=0)
    q = jnp.sum(x * x, axis=0)
    v_ref[...] = s * lax.rsqrt(q)


def _sc_body(ids_t_hbm, ids_f_hbm, v_hbm, out_t, out_hf, out_tf,
             idx_v, s_v, ot_v, ohf_v, otf_v, sem, sem2):
    w = lax.axis_index("s") * NC + lax.axis_index("c")

    staged = []
    for k in range(3):
        staged.append(pltpu.async_copy(ids_t_hbm.at[k, pl.ds(w * RPW, RPW)],
                                       idx_v.at[pl.ds(k * RPW, RPW)], sem2))
    for k in range(2):
        staged.append(pltpu.async_copy(ids_f_hbm.at[k, pl.ds(w * RPW, RPW)],
                                       idx_v.at[pl.ds((3 + k) * RPW, RPW)],
                                       sem2))
    for cp in staged:
        cp.wait()

    gathers = [
        pltpu.async_copy(v_hbm.at[idx_v.at[c]],
                         s_v.at[pl.ds(c * CHUNK, CHUNK)], sem)
        for c in range(NCHUNK)
    ]
    for cp in gathers:
        cp.wait()

    def comb_body(i, carry):
        o = i * 16
        sh = s_v[pl.ds(o, 16)]
        sr = s_v[pl.ds(BPW + o, 16)]
        st = s_v[pl.ds(2 * BPW + o, 16)]
        shf = s_v[pl.ds(3 * BPW + o, 16)]
        stf = s_v[pl.ds(4 * BPW + o, 16)]
        ot_v[pl.ds(o, 16)] = sh + sr - st
        ohf_v[pl.ds(o, 16)] = shf + sr - st
        otf_v[pl.ds(o, 16)] = sh + sr - stf
        return carry

    lax.fori_loop(0, BPW // 16, comb_body, 0)

    base = w * BPW
    pltpu.sync_copy(ot_v, out_t.at[pl.ds(base, BPW)])
    pltpu.sync_copy(ohf_v, out_hf.at[pl.ds(base, BPW)])
    pltpu.sync_copy(otf_v, out_tf.at[pl.ds(base, BPW)])


def kernel(ids_true_batch, ids_false_batch, ent_table):
    tbl_t = ent_table.T

    grid = pl.cdiv(NUM_ENT, LBLK)
    v = pl.pallas_call(
        _tc_body,
        grid=(grid,),
        in_specs=[pl.BlockSpec((EMB_DIM, LBLK), lambda i: (0, i))],
        out_specs=pl.BlockSpec((LBLK,), lambda i: (i,)),
        out_shape=jax.ShapeDtypeStruct((NUM_ENT,), jnp.float32),
    )(tbl_t)

    ids_t = ids_true_batch.astype(jnp.int32).reshape(3, BATCH // CHUNK, CHUNK)
    ids_f = ids_false_batch.astype(jnp.int32).reshape(2, BATCH // CHUNK, CHUNK)

    mesh = plsc.VectorSubcoreMesh(core_axis_name="c", subcore_axis_name="s")
    fn = pl.kernel(
        _sc_body,
        mesh=mesh,
        compiler_params=pltpu.CompilerParams(
            needs_layout_passes=False, use_tc_tiling_on_sc=False
        ),
        out_type=[jax.ShapeDtypeStruct((BATCH,), jnp.float32)] * 3,
        scratch_types=[
            pltpu.VMEM((NCHUNK, CHUNK), jnp.int32),
            pltpu.VMEM((NIDS * BPW,), jnp.float32),
            pltpu.VMEM((BPW,), jnp.float32),
            pltpu.VMEM((BPW,), jnp.float32),
            pltpu.VMEM((BPW,), jnp.float32),
            pltpu.SemaphoreType.DMA,
            pltpu.SemaphoreType.DMA,
        ],
    )
    t, hf, tf = fn(ids_t, ids_f, v)
    return (t, hf, tf)
